# baseline (speedup 1.0000x reference)
import functools

import jax
import jax.numpy as jnp
from jax import lax
from jax.experimental import pallas as pl
from jax.experimental.pallas import tpu as pltpu
from jax.experimental.pallas import tpu_sc as plsc

NC, NS, L = 2, 16, 16
L2 = 2 * L
NW = NC * NS
N = 16384
F = 256
ROWS = 64
D = 4 * F
DW = D // 2
B_W = N // NW
B_HALF = 32
N_PAIRS = B_W // (2 * B_HALF)


def _emb_body(boxes_hbm, w_hbm, out_hbm, table_v, boxes_v,
              out0_v, out1_v, loff_v, roff_v, lw_v, rw_v, sem0, sem1):
    wid = lax.axis_index("s") * NC + lax.axis_index("c")
    pltpu.sync_copy(w_hbm, table_v)
    pltpu.sync_copy(boxes_hbm.at[pl.ds(wid * (B_W * 4), B_W * 4)], boxes_v)
    lane = lax.iota(jnp.int32, L)
    lane4 = lane * 4

    def stage_body(blk, carry):
        for j in range(4):
            pos = lane4 + (blk * 4 * L + j)
            bx = plsc.load_gather(boxes_v, [pos])
            data = bx * 16.0
            li = jnp.clip(data, 0.0, 15.0).astype(jnp.int32)
            ri = jnp.minimum(li + 1, 15)
            lw = data - li.astype(jnp.float32)
            plsc.store_scatter(loff_v, [pos], li * (D // 2) + (j * F // 2))
            plsc.store_scatter(roff_v, [pos], ri * (D // 2) + (j * F // 2))
            plsc.store_scatter(lw_v, [pos], lw)
            plsc.store_scatter(rw_v, [pos], 1.0 - lw)
        return carry

    lax.fori_loop(0, B_W // L, stage_body, 0)

    bufs = (out0_v, out1_v)
    sems = (sem0, sem1)

    def pair_body(ci, carry):
        for h in range(2):
            buf, sem = bufs[h], sems[h]
            idx = ci * 2 + h

            @pl.when(ci > 0)
            def _wait_prev():
                pltpu.make_async_copy(
                    buf, out_hbm.at[pl.ds(0, B_HALF * DW)], sem).wait()

            kbase = idx * B_HALF * 4

            @plsc.parallel_loop(0, B_HALF, unroll=2)
            def n_body(n):
                k = kbase + n * 4
                lv = loff_v[pl.ds(k, L)]
                rv = roff_v[pl.ds(k, L)]
                lwv4 = lw_v[pl.ds(k, L)]
                rwv4 = rw_v[pl.ds(k, L)]
                obase = n * DW
                for j in range(4):
                    lo = lv[j]
                    ro = rv[j]
                    lwf = jnp.full((L,), lwv4[j], jnp.float32)
                    rwf = jnp.full((L,), rwv4[j], jnp.float32)
                    lwb = plsc.pack(lwf, lwf,
                                    format=plsc.PackFormat.INTERLEAVED)
                    rwb = plsc.pack(rwf, rwf,
                                    format=plsc.PackFormat.INTERLEAVED)
                    ob = obase + j * (F // 2)
                    for t in range(F // L2):
                        gl = plsc.bitcast(
                            table_v[pl.ds(lo + t * L, L)], jnp.bfloat16)
                        gr = plsc.bitcast(
                            table_v[pl.ds(ro + t * L, L)], jnp.bfloat16)
                        res = rwb * gl + lwb * gr
                        bufs[h][pl.ds(ob + t * L, L)] = plsc.bitcast(
                            res, jnp.float32)

            hbase = (wid * B_W + idx * B_HALF) * DW
            pltpu.async_copy(buf, out_hbm.at[pl.ds(hbase, B_HALF * DW)], sem)
        return carry

    lax.fori_loop(0, N_PAIRS, pair_body, 0)
    for h in range(2):
        pltpu.make_async_copy(
            bufs[h], out_hbm.at[pl.ds(0, B_HALF * DW)], sems[h]).wait()


_emb_call = functools.partial(
    pl.kernel,
    out_type=jax.ShapeDtypeStruct((N * DW,), jnp.float32),
    mesh=plsc.VectorSubcoreMesh(core_axis_name="c", subcore_axis_name="s"),
    compiler_params=pltpu.CompilerParams(
        needs_layout_passes=False, disable_bounds_checks=True),
    scratch_types=[
        pltpu.VMEM((ROWS * F // 2,), jnp.float32),
        pltpu.VMEM((B_W * 4,), jnp.float32),
        pltpu.VMEM((B_HALF * DW,), jnp.float32),
        pltpu.VMEM((B_HALF * DW,), jnp.float32),
        pltpu.VMEM((4 * B_W,), jnp.int32),
        pltpu.VMEM((4 * B_W,), jnp.int32),
        pltpu.VMEM((4 * B_W,), jnp.float32),
        pltpu.VMEM((4 * B_W,), jnp.float32),
        pltpu.SemaphoreType.DMA,
        pltpu.SemaphoreType.DMA,
    ],
)(_emb_body)


@jax.jit
def kernel(boxes, pos_weight):
    table_bf16 = pos_weight.astype(jnp.bfloat16).reshape(-1, 2)
    table_words = jax.lax.bitcast_convert_type(table_bf16, jnp.float32)
    out_words = _emb_call(boxes.reshape(-1), table_words.reshape(-1))
    out_bf16 = jax.lax.bitcast_convert_type(out_words, jnp.bfloat16)
    return out_bf16.reshape(N, D).astype(jnp.float32)


# speedup vs baseline: 1.6529x; 1.6529x over previous
import functools

import jax
import jax.numpy as jnp
from jax import lax
from jax.experimental import pallas as pl
from jax.experimental.pallas import tpu as pltpu
from jax.experimental.pallas import tpu_sc as plsc

NC, NS, L = 2, 16, 16
NW = NC * NS
N = 16384
F = 256
ROWS = 64
D = 4 * F
B_W = N // NW
B_HALF = 32
N_PAIRS = B_W // (2 * B_HALF)


def _emb_body(boxes_hbm, w_hbm, out_hbm, table_v, boxes_v,
              out0_v, out1_v, loff_v, roff_v, lw_v, rw_v, sem0, sem1):
    wid = lax.axis_index("s") * NC + lax.axis_index("c")
    pltpu.sync_copy(w_hbm, table_v)
    pltpu.sync_copy(boxes_hbm.at[pl.ds(wid * (B_W * 4), B_W * 4)], boxes_v)
    lane = lax.iota(jnp.int32, L)
    lane4 = lane * 4

    def stage_body(blk, carry):
        for j in range(4):
            pos = lane4 + (blk * 4 * L + j)
            bx = plsc.load_gather(boxes_v, [pos])
            data = bx * 16.0
            li = jnp.clip(data, 0.0, 15.0).astype(jnp.int32)
            ri = jnp.minimum(li + 1, 15)
            lw = data - li.astype(jnp.float32)
            plsc.store_scatter(loff_v, [pos], li * D + (j * F))
            plsc.store_scatter(roff_v, [pos], ri * D + (j * F))
            plsc.store_scatter(lw_v, [pos], lw)
            plsc.store_scatter(rw_v, [pos], 1.0 - lw)
        return carry

    lax.fori_loop(0, B_W // L, stage_body, 0)

    bufs = (out0_v, out1_v)
    sems = (sem0, sem1)

    def pair_body(ci, carry):
        for h in range(2):
            buf, sem = bufs[h], sems[h]
            idx = ci * 2 + h

            @pl.when(ci > 0)
            def _wait_prev():
                pltpu.make_async_copy(
                    buf, out_hbm.at[pl.ds(0, B_HALF * D)], sem).wait()

            kbase = idx * B_HALF * 4

            @plsc.parallel_loop(0, B_HALF, unroll=4)
            def n_body(n):
                k = kbase + n * 4
                lv = loff_v[pl.ds(k, L)]
                rv = roff_v[pl.ds(k, L)]
                lwv4 = lw_v[pl.ds(k, L)]
                rwv4 = rw_v[pl.ds(k, L)]
                obase = n * D
                for j in range(4):
                    lo = lv[j]
                    ro = rv[j]
                    lwv = jnp.full((L,), lwv4[j], jnp.float32)
                    rwv = jnp.full((L,), rwv4[j], jnp.float32)
                    ob = obase + j * F
                    for t in range(F // L):
                        gl = table_v[pl.ds(lo + t * L, L)]
                        gr = table_v[pl.ds(ro + t * L, L)]
                        out = rwv * gl + lwv * gr
                        bufs[h][pl.ds(ob + t * L, L)] = out

            hbase = (wid * B_W + idx * B_HALF) * D
            pltpu.async_copy(buf, out_hbm.at[pl.ds(hbase, B_HALF * D)], sem)
        return carry

    lax.fori_loop(0, N_PAIRS, pair_body, 0)
    for h in range(2):
        pltpu.make_async_copy(
            bufs[h], out_hbm.at[pl.ds(0, B_HALF * D)], sems[h]).wait()


_emb_call = functools.partial(
    pl.kernel,
    out_type=jax.ShapeDtypeStruct((N * D,), jnp.float32),
    mesh=plsc.VectorSubcoreMesh(core_axis_name="c", subcore_axis_name="s"),
    compiler_params=pltpu.CompilerParams(
        needs_layout_passes=False, disable_bounds_checks=True),
    scratch_types=[
        pltpu.VMEM((ROWS * F,), jnp.float32),
        pltpu.VMEM((B_W * 4,), jnp.float32),
        pltpu.VMEM((B_HALF * D,), jnp.float32),
        pltpu.VMEM((B_HALF * D,), jnp.float32),
        pltpu.VMEM((4 * B_W,), jnp.int32),
        pltpu.VMEM((4 * B_W,), jnp.int32),
        pltpu.VMEM((4 * B_W,), jnp.float32),
        pltpu.VMEM((4 * B_W,), jnp.float32),
        pltpu.SemaphoreType.DMA,
        pltpu.SemaphoreType.DMA,
    ],
)(_emb_body)


@jax.jit
def kernel(boxes, pos_weight):
    out = _emb_call(boxes.reshape(-1), pos_weight.reshape(-1))
    return out.reshape(N, D)
